# Initial kernel scaffold; baseline (speedup 1.0000x reference)
import jax
import jax.numpy as jnp
from jax.experimental import pallas as pl


def kernel(position_ids, cos_emb, sin_emb):
    raise NotImplementedError("write your pallas kernel here")


# speedup vs baseline: 1.5142x; 1.5142x over previous
import functools

import jax
import jax.numpy as jnp
from jax import lax
from jax.experimental import pallas as pl
from jax.experimental.pallas import tpu as pltpu
from jax.experimental.pallas import tpu_sc as plsc

HEAD_DIM = 128
CHUNK = 128


def _rope_gather_fn(N, chunks_per_w, NC):
    mesh = plsc.VectorSubcoreMesh(core_axis_name="c", subcore_axis_name="s")

    @functools.partial(
        pl.kernel,
        mesh=mesh,
        out_type=(
            jax.ShapeDtypeStruct((N, HEAD_DIM), jnp.float32),
            jax.ShapeDtypeStruct((N, HEAD_DIM), jnp.float32),
        ),
        scratch_types=[
            pltpu.VMEM((chunks_per_w, CHUNK), jnp.int32),
            pltpu.VMEM((CHUNK, HEAD_DIM), jnp.float32),
            pltpu.VMEM((CHUNK, HEAD_DIM), jnp.float32),
            pltpu.SemaphoreType.DMA,
            pltpu.SemaphoreType.DMA,
        ],
    )
    def body(idx_hbm, cos_hbm, sin_hbm, cos_out, sin_out,
             idx_v, cos_v, sin_v, sem_c, sem_s):
        wid = lax.axis_index("s") * NC + lax.axis_index("c")
        row0 = wid * chunks_per_w
        pltpu.sync_copy(idx_hbm.at[pl.ds(row0, chunks_per_w)], idx_v)
        for j in range(chunks_per_w):
            cc = pltpu.async_copy(cos_hbm.at[idx_v.at[j]], cos_v, sem_c)
            sc = pltpu.async_copy(sin_hbm.at[idx_v.at[j]], sin_v, sem_s)
            cc.wait()
            sc.wait()
            base = (row0 + j) * CHUNK
            pltpu.sync_copy(cos_v, cos_out.at[pl.ds(base, CHUNK)])
            pltpu.sync_copy(sin_v, sin_out.at[pl.ds(base, CHUNK)])

    return body


def kernel(position_ids, cos_emb, sin_emb):
    B, S = position_ids.shape
    N = B * S
    info = plsc.get_sparse_core_info()
    NC, NS = info.num_cores, info.num_subcores
    NW = NC * NS
    chunks_total = N // CHUNK
    chunks_per_w = chunks_total // NW

    idx = position_ids.reshape(chunks_total, CHUNK).astype(jnp.int32)
    cos_flat, sin_flat = _rope_gather_fn(N, chunks_per_w, NC)(
        idx, cos_emb, sin_emb)
    return (cos_flat.reshape(B, S, HEAD_DIM),
            sin_flat.reshape(B, S, HEAD_DIM))


# speedup vs baseline: 1.6373x; 1.0813x over previous
import functools

import jax
import jax.numpy as jnp
from jax import lax
from jax.experimental import pallas as pl
from jax.experimental.pallas import tpu as pltpu
from jax.experimental.pallas import tpu_sc as plsc

HEAD_DIM = 128
CHUNK = 128


def _rope_gather_fn(N, chunks_per_w, NC):
    mesh = plsc.VectorSubcoreMesh(core_axis_name="c", subcore_axis_name="s")

    NBUF = 2

    @functools.partial(
        pl.kernel,
        mesh=mesh,
        out_type=(
            jax.ShapeDtypeStruct((N, HEAD_DIM), jnp.float32),
            jax.ShapeDtypeStruct((N, HEAD_DIM), jnp.float32),
        ),
        scratch_types=[
            pltpu.VMEM((chunks_per_w, CHUNK), jnp.int32),
            pltpu.VMEM((NBUF, CHUNK, HEAD_DIM), jnp.float32),
            pltpu.VMEM((NBUF, CHUNK, HEAD_DIM), jnp.float32),
        ]
        + [pltpu.SemaphoreType.DMA] * (2 * NBUF),
    )
    def body(idx_hbm, cos_hbm, sin_hbm, cos_out, sin_out,
             idx_v, cos_v, sin_v, *sems):
        gsem, wsem = sems[:NBUF], sems[NBUF:]
        wid = lax.axis_index("s") * NC + lax.axis_index("c")
        row0 = wid * chunks_per_w
        pltpu.sync_copy(idx_hbm.at[pl.ds(row0, chunks_per_w)], idx_v)

        def issue_gather(j):
            b = j % NBUF
            return (
                pltpu.async_copy(cos_hbm.at[idx_v.at[j]], cos_v.at[b], gsem[b]),
                pltpu.async_copy(sin_hbm.at[idx_v.at[j]], sin_v.at[b], gsem[b]),
            )

        pending_w = [None] * NBUF
        g = issue_gather(0)
        for j in range(chunks_per_w):
            b = j % NBUF
            nb = (j + 1) % NBUF
            g_next = None
            if j + 1 < chunks_per_w:
                if pending_w[nb] is not None:
                    for d in pending_w[nb]:
                        d.wait()
                    pending_w[nb] = None
                g_next = issue_gather(j + 1)
            for d in g:
                d.wait()
            base = (row0 + j) * CHUNK
            pending_w[b] = (
                pltpu.async_copy(cos_v.at[b], cos_out.at[pl.ds(base, CHUNK)],
                                 wsem[b]),
                pltpu.async_copy(sin_v.at[b], sin_out.at[pl.ds(base, CHUNK)],
                                 wsem[b]),
            )
            g = g_next
        for w in pending_w:
            if w is not None:
                for d in w:
                    d.wait()

    return body


def kernel(position_ids, cos_emb, sin_emb):
    B, S = position_ids.shape
    N = B * S
    info = plsc.get_sparse_core_info()
    NC, NS = info.num_cores, info.num_subcores
    NW = NC * NS
    chunks_total = N // CHUNK
    chunks_per_w = chunks_total // NW

    idx = position_ids.reshape(chunks_total, CHUNK).astype(jnp.int32)
    cos_flat, sin_flat = _rope_gather_fn(N, chunks_per_w, NC)(
        idx, cos_emb, sin_emb)
    return (cos_flat.reshape(B, S, HEAD_DIM),
            sin_flat.reshape(B, S, HEAD_DIM))


# speedup vs baseline: 1.6811x; 1.0267x over previous
import functools

import jax
import jax.numpy as jnp
from jax import lax
from jax.experimental import pallas as pl
from jax.experimental.pallas import tpu as pltpu
from jax.experimental.pallas import tpu_sc as plsc

HEAD_DIM = 128
CHUNK = 128


def _rope_gather_fn(N, chunks_per_w, NC):
    mesh = plsc.VectorSubcoreMesh(core_axis_name="c", subcore_axis_name="s")
    NBUF = 3

    @functools.partial(
        pl.kernel,
        mesh=mesh,
        out_type=(
            jax.ShapeDtypeStruct((N, HEAD_DIM), jnp.float32),
            jax.ShapeDtypeStruct((N, HEAD_DIM), jnp.float32),
        ),
        scratch_types=[
            pltpu.VMEM((chunks_per_w, CHUNK), jnp.int32),
            pltpu.VMEM((NBUF, CHUNK, HEAD_DIM), jnp.float32),
            pltpu.VMEM((NBUF, CHUNK, HEAD_DIM), jnp.float32),
        ]
        + [pltpu.SemaphoreType.DMA] * (2 * NBUF),
    )
    def body(idx_hbm, cos_hbm, sin_hbm, cos_out, sin_out,
             idx_v, cos_v, sin_v, *sems):
        gsem, wsem = sems[:NBUF], sems[NBUF:]
        wid = lax.axis_index("s") * NC + lax.axis_index("c")
        row0 = wid * chunks_per_w
        pltpu.sync_copy(idx_hbm.at[pl.ds(row0, chunks_per_w)], idx_v)

        def issue_gather(j):
            b = j % NBUF
            return (
                pltpu.async_copy(cos_hbm.at[idx_v.at[j]], cos_v.at[b], gsem[b]),
                pltpu.async_copy(sin_hbm.at[idx_v.at[j]], sin_v.at[b], gsem[b]),
            )

        pending_g = [None] * NBUF
        pending_w = [None] * NBUF
        for j in range(min(NBUF - 1, chunks_per_w)):
            pending_g[j % NBUF] = issue_gather(j)
        for j in range(chunks_per_w):
            b = j % NBUF
            jn = j + NBUF - 1
            if jn < chunks_per_w:
                nb = jn % NBUF
                if pending_w[nb] is not None:
                    for d in pending_w[nb]:
                        d.wait()
                    pending_w[nb] = None
                pending_g[nb] = issue_gather(jn)
            for d in pending_g[b]:
                d.wait()
            pending_g[b] = None
            base = (row0 + j) * CHUNK
            pending_w[b] = (
                pltpu.async_copy(cos_v.at[b], cos_out.at[pl.ds(base, CHUNK)],
                                 wsem[b]),
                pltpu.async_copy(sin_v.at[b], sin_out.at[pl.ds(base, CHUNK)],
                                 wsem[b]),
            )
        for w in pending_w:
            if w is not None:
                for d in w:
                    d.wait()

    return body


def kernel(position_ids, cos_emb, sin_emb):
    B, S = position_ids.shape
    N = B * S
    info = plsc.get_sparse_core_info()
    NC, NS = info.num_cores, info.num_subcores
    NW = NC * NS
    chunks_total = N // CHUNK
    chunks_per_w = chunks_total // NW

    idx = position_ids.reshape(chunks_total, CHUNK).astype(jnp.int32)
    cos_flat, sin_flat = _rope_gather_fn(N, chunks_per_w, NC)(
        idx, cos_emb, sin_emb)
    return (cos_flat.reshape(B, S, HEAD_DIM),
            sin_flat.reshape(B, S, HEAD_DIM))
